# baseline (speedup 1.0000x reference)
import functools

import jax
import jax.numpy as jnp
from jax.experimental import pallas as pl
from jax.experimental.pallas import tpu as pltpu
from jax.experimental.pallas import tpu_sc as plsc

_N = 10000
_E = 160000
_D = 128
_R = 4
_DW = _D // 2

_NC = 2
_NS = 16
_B = 128
_NB_TILE = 40
_NB_PHASE = 20
_E_PAD = _NC * _NS * _NB_TILE * _B
_ACC_ROWS = 10240
_ROWS_PER_SUB = _ACC_ROWS // _NS
_DEPTH = 2


def _rne_bf16_bits(v):
    u = jax.lax.bitcast_convert_type(v, jnp.uint32)
    r = u + jnp.uint32(0x7FFF) + ((u >> jnp.uint32(16)) & jnp.uint32(1))
    return r >> jnp.uint32(16)


def _mm_body(x_ref, w_ref, oself_ref, omsg_ref):
    r = pl.program_id(1)
    y = jnp.dot(x_ref[...], w_ref[0], preferred_element_type=jnp.float32)

    @pl.when(r == 0)
    def _():
        oself_ref[...] = y

    @pl.when(r > 0)
    def _():
        lo = _rne_bf16_bits(y[:, :_DW])
        hi = _rne_bf16_bits(y[:, _DW:])
        omsg_ref[...] = jax.lax.bitcast_convert_type(
            lo | (hi << jnp.uint32(16)), jnp.int32)


def _matmul_all(x, wcat):
    nbx = 5
    bn = _N // nbx
    return pl.pallas_call(
        _mm_body,
        grid=(nbx, _R + 1),
        in_specs=[
            pl.BlockSpec((bn, _D), lambda i, r: (i, 0)),
            pl.BlockSpec((1, _D, _D), lambda i, r: (r, 0, 0)),
        ],
        out_specs=[
            pl.BlockSpec((bn, _D), lambda i, r: (i, 0)),
            pl.BlockSpec((bn, _DW),
                         lambda i, r: (jnp.maximum(r - 1, 0) * nbx + i, 0)),
        ],
        out_shape=[jax.ShapeDtypeStruct((_N, _D), jnp.float32),
                   jax.ShapeDtypeStruct((_R * _N, _DW), jnp.int32)],
    )(x, wcat)


_PERM = sum([list(range(g * 32, g * 32 + 16)) for g in range(4)], []) + \
        sum([list(range(g * 32 + 16, g * 32 + 32)) for g in range(4)], [])


def _gidx_body(src_ref, et_ref, o_ref):
    o_ref[...] = et_ref[...] * _N + src_ref[...]


def _make_gidx(src2d, et2d):
    nrows = _E_PAD // _B
    return pl.pallas_call(
        _gidx_body,
        grid=(8,),
        in_specs=[
            pl.BlockSpec((nrows // 8, _B), lambda i: (i, 0)),
            pl.BlockSpec((nrows // 8, _B), lambda i: (i, 0)),
        ],
        out_specs=pl.BlockSpec((nrows // 8, _B), lambda i: (i, 0)),
        out_shape=jax.ShapeDtypeStruct((nrows, _B), jnp.int32),
    )(src2d, et2d)


def _combine_body(y_ref, a0_ref, a1_ref, x_ref, b_ref, o_ref, *, residual):
    o = y_ref[...] + a0_ref[...] + a1_ref[...] + b_ref[...]
    if residual:
        o = o + x_ref[...]
    o_ref[...] = jnp.maximum(o, 0.0)


def _combine(selfy, agg0, agg1, x, b2d, residual):
    nbx = 5
    bn = _N // nbx
    return pl.pallas_call(
        functools.partial(_combine_body, residual=residual),
        grid=(nbx,),
        in_specs=[
            pl.BlockSpec((bn, _D), lambda i: (i, 0)),
            pl.BlockSpec((bn, _D), lambda i: (i, 0)),
            pl.BlockSpec((bn, _D), lambda i: (i, 0)),
            pl.BlockSpec((bn, _D), lambda i: (i, 0)),
            pl.BlockSpec((1, _D), lambda i: (0, 0)),
        ],
        out_specs=pl.BlockSpec((bn, _D), lambda i: (i, 0)),
        out_shape=jax.ShapeDtypeStruct((_N, _D), jnp.float32),
    )(selfy, agg0, agg1, x, b2d)


def _sc_body(ytab, gidx_hbm, dst_hbm, out0, out1,
             acc, gidx_v, dst_v, rows_i, frow, *sems):
    c = jax.lax.axis_index("c")
    s = jax.lax.axis_index("s")
    wid = c * _NS + s

    def _zero_row(i, _):
        for k in range(_D // 16):
            frow[i, pl.ds(k * 16, 16)] = jnp.zeros((16,), jnp.float32)
        return 0
    jax.lax.fori_loop(0, _B, _zero_row, 0)
    base = s * _ROWS_PER_SUB
    for j in range(_ROWS_PER_SUB // _B):
        pltpu.sync_copy(frow, acc.at[pl.ds(base + j * _B, _B)])
    plsc.subcore_barrier()

    def _convert(k):
        mask = jnp.full((16,), -65536, jnp.int32)

        def _row(i, _):
            for g in range(_D // 32):
                w = rows_i[k, i, pl.ds(g * 16, 16)]
                frow[i, pl.ds(g * 32, 16)] = plsc.bitcast(
                    jax.lax.shift_left(w, 16), jnp.float32)
                frow[i, pl.ds(g * 32 + 16, 16)] = plsc.bitcast(
                    jax.lax.bitwise_and(w, mask), jnp.float32)
            return 0
        jax.lax.fori_loop(0, _B, _row, 0)

    def _start(b, k):
        pltpu.async_copy(ytab.at[gidx_v.at[b]], rows_i.at[k], sems[k])

    def _wait(b, k):
        pltpu.make_async_copy(ytab.at[gidx_v.at[b]], rows_i.at[k],
                              sems[k]).wait()

    for p in range(_NB_TILE // _NB_PHASE):
        row0 = wid * _NB_TILE + p * _NB_PHASE
        pltpu.sync_copy(gidx_hbm.at[pl.ds(row0, _NB_PHASE)], gidx_v)
        pltpu.sync_copy(dst_hbm.at[pl.ds(row0, _NB_PHASE)], dst_v)

        for k in range(_DEPTH):
            _start(k, k)

        def _step(i, _):
            for k in range(_DEPTH):
                b = i * _DEPTH + k
                _wait(b, k)
                _convert(k)
                pltpu.sync_copy(frow, acc.at[dst_v.at[b]], add=True)
                nxt = b + _DEPTH

                @pl.when(nxt < _NB_PHASE)
                def _():
                    _start(nxt, k)
            return 0
        jax.lax.fori_loop(0, _NB_PHASE // _DEPTH, _step, 0)
    plsc.subcore_barrier()

    for cc, out_ref in ((0, out0), (1, out1)):
        @pl.when(c == cc)
        def _():
            @pl.when(s < _NS - 1)
            def _():
                r0 = pl.multiple_of(s * 624, 8)
                pltpu.sync_copy(acc.at[pl.ds(r0, 624)],
                                out_ref.at[pl.ds(r0, 624)])

            @pl.when(s == _NS - 1)
            def _():
                pltpu.sync_copy(acc.at[pl.ds(9360, 640)],
                                out_ref.at[pl.ds(9360, 640)])


def _sc_aggregate(ytab, gidx2d, dst2d):
    mesh = plsc.VectorSubcoreMesh(core_axis_name="c", subcore_axis_name="s")
    f = pl.kernel(
        _sc_body,
        out_type=[jax.ShapeDtypeStruct((_N, _D), jnp.float32),
                  jax.ShapeDtypeStruct((_N, _D), jnp.float32)],
        mesh=mesh,
        compiler_params=pltpu.CompilerParams(use_tc_tiling_on_sc=False,
                                             needs_layout_passes=False),
        scratch_types=[
            pltpu.VMEM_SHARED((_ACC_ROWS, _D), jnp.float32),
            pltpu.VMEM((_NB_PHASE, _B), jnp.int32),
            pltpu.VMEM((_NB_PHASE, _B), jnp.int32),
            pltpu.VMEM((_DEPTH, _B, _DW), jnp.int32),
            pltpu.VMEM((_B, _D), jnp.float32),
        ] + [pltpu.SemaphoreType.DMA] * _DEPTH,
    )
    return f(ytab, gidx2d, dst2d)


@jax.jit
def kernel(x, edge_index, edge_type, W1, S1, b1, W2, S2, b2):
    src = edge_index[0].astype(jnp.int32)
    dst = edge_index[1].astype(jnp.int32)
    et = edge_type.astype(jnp.int32)

    pad = _E_PAD - _E
    src2d = jnp.concatenate(
        [src, jnp.zeros((pad,), jnp.int32)]).reshape(_E_PAD // _B, _B)
    et2d = jnp.concatenate(
        [et, jnp.zeros((pad,), jnp.int32)]).reshape(_E_PAD // _B, _B)
    dst2d = jnp.concatenate(
        [dst, jnp.full((pad,), _N, jnp.int32)]).reshape(_E_PAD // _B, _B)
    gidx2d = _make_gidx(src2d, et2d)

    b1r = b1.reshape(1, _D)
    b2r = b2.reshape(1, _D)

    perm = jnp.array(_PERM)
    self1, ytab1 = _matmul_all(
        x, jnp.concatenate([S1[None], W1[:, :, perm]], axis=0))
    agg0, agg1 = _sc_aggregate(ytab1, gidx2d, dst2d)
    h = _combine(self1, agg0, agg1, x, b1r, residual=True)

    self2, ytab2 = _matmul_all(
        h, jnp.concatenate([S2[None], W2[:, :, perm]], axis=0))
    agg0b, agg1b = _sc_aggregate(ytab2, gidx2d, dst2d)
    out = _combine(self2, agg0b, agg1b, h, b2r, residual=False)
    return out


# speedup vs baseline: 1.1270x; 1.1270x over previous
import functools

import jax
import jax.numpy as jnp
from jax.experimental import pallas as pl
from jax.experimental.pallas import tpu as pltpu
from jax.experimental.pallas import tpu_sc as plsc

_N = 10000
_E = 160000
_D = 128
_R = 4
_DW = _D // 2

_NC = 2
_NS = 16
_B = 128
_NB_TILE = 40
_NB_PHASE = 20
_E_PAD = _NC * _NS * _NB_TILE * _B
_ACC_ROWS = 10240
_ROWS_PER_SUB = _ACC_ROWS // _NS
_DEPTH = 2
_HB = _B // 2


def _self_body(x_ref, w_ref, o_ref):
    o_ref[...] = jnp.dot(x_ref[...], w_ref[...],
                         preferred_element_type=jnp.float32)


def _matmul_self(x, s):
    nbx = 5
    bn = _N // nbx
    return pl.pallas_call(
        _self_body,
        grid=(nbx,),
        in_specs=[
            pl.BlockSpec((bn, _D), lambda i: (i, 0)),
            pl.BlockSpec((_D, _D), lambda i: (0, 0)),
        ],
        out_specs=pl.BlockSpec((bn, _D), lambda i: (i, 0)),
        out_shape=jax.ShapeDtypeStruct((_N, _D), jnp.float32),
    )(x, s)


def _rne_bf16_bits(v):
    u = jax.lax.bitcast_convert_type(v, jnp.uint32)
    r = u + jnp.uint32(0x7FFF) + ((u >> jnp.uint32(16)) & jnp.uint32(1))
    return r >> jnp.uint32(16)


def _msg_body(x_ref, w_ref, o_ref):
    y = jnp.dot(x_ref[...], w_ref[0], preferred_element_type=jnp.float32)
    lo = _rne_bf16_bits(y[:, :_DW])
    hi = _rne_bf16_bits(y[:, _DW:])
    o_ref[...] = jax.lax.bitcast_convert_type(
        lo | (hi << jnp.uint32(16)), jnp.int32)


def _matmul_msgs(x, w):
    nbx = 5
    bn = _N // nbx
    return pl.pallas_call(
        _msg_body,
        grid=(_R, nbx),
        in_specs=[
            pl.BlockSpec((bn, _D), lambda r, i: (i, 0)),
            pl.BlockSpec((1, _D, _D), lambda r, i: (r, 0, 0)),
        ],
        out_specs=pl.BlockSpec((bn, _DW), lambda r, i: (r * nbx + i, 0)),
        out_shape=jax.ShapeDtypeStruct((_R * _N, _DW), jnp.int32),
    )(x, w)


_PERM = sum([list(range(g * 32, g * 32 + 16)) for g in range(4)], []) + \
        sum([list(range(g * 32 + 16, g * 32 + 32)) for g in range(4)], [])


def _gidx_body(src_ref, et_ref, o_ref):
    o_ref[...] = et_ref[...] * _N + src_ref[...]


def _make_gidx(src2d, et2d):
    nrows = _E_PAD // _B
    return pl.pallas_call(
        _gidx_body,
        grid=(8,),
        in_specs=[
            pl.BlockSpec((nrows // 8, _B), lambda i: (i, 0)),
            pl.BlockSpec((nrows // 8, _B), lambda i: (i, 0)),
        ],
        out_specs=pl.BlockSpec((nrows // 8, _B), lambda i: (i, 0)),
        out_shape=jax.ShapeDtypeStruct((nrows, _B), jnp.int32),
    )(src2d, et2d)


def _combine_body(y_ref, a0_ref, a1_ref, x_ref, b_ref, o_ref, *, residual):
    o = y_ref[...] + a0_ref[...] + a1_ref[...] + b_ref[...]
    if residual:
        o = o + x_ref[...]
    o_ref[...] = jnp.maximum(o, 0.0)


def _combine(selfy, agg0, agg1, x, b2d, residual):
    nbx = 5
    bn = _N // nbx
    return pl.pallas_call(
        functools.partial(_combine_body, residual=residual),
        grid=(nbx,),
        in_specs=[
            pl.BlockSpec((bn, _D), lambda i: (i, 0)),
            pl.BlockSpec((bn, _D), lambda i: (i, 0)),
            pl.BlockSpec((bn, _D), lambda i: (i, 0)),
            pl.BlockSpec((bn, _D), lambda i: (i, 0)),
            pl.BlockSpec((1, _D), lambda i: (0, 0)),
        ],
        out_specs=pl.BlockSpec((bn, _D), lambda i: (i, 0)),
        out_shape=jax.ShapeDtypeStruct((_N, _D), jnp.float32),
    )(selfy, agg0, agg1, x, b2d)


def _sc_body(ytab, gidx_hbm, dst_hbm, out0, out1,
             acc, gidx_v, dst_v, rows_i, frow2, *allsems):
    sems = allsems[:_DEPTH]
    ssems = allsems[_DEPTH:]
    c = jax.lax.axis_index("c")
    s = jax.lax.axis_index("s")
    wid = c * _NS + s

    def _zero_row(i, _):
        for k in range(_D // 16):
            frow2[0, i, pl.ds(k * 16, 16)] = jnp.zeros((16,), jnp.float32)
        return 0
    jax.lax.fori_loop(0, _HB, _zero_row, 0)
    base = s * _ROWS_PER_SUB
    for j in range(_ROWS_PER_SUB // _HB):
        pltpu.sync_copy(frow2.at[0], acc.at[pl.ds(base + j * _HB, _HB)])
    plsc.subcore_barrier()

    def _convert_half(k, h):
        mask = jnp.full((16,), -65536, jnp.int32)

        def _row(i, _):
            ri = h * _HB + i
            for g in range(_D // 32):
                w = rows_i[k, ri, pl.ds(g * 16, 16)]
                frow2[h, i, pl.ds(g * 32, 16)] = plsc.bitcast(
                    jax.lax.shift_left(w, 16), jnp.float32)
                frow2[h, i, pl.ds(g * 32 + 16, 16)] = plsc.bitcast(
                    jax.lax.bitwise_and(w, mask), jnp.float32)
            return 0
        jax.lax.fori_loop(0, _HB, _row, 0)

    def _wait_scatter(h):
        pltpu.make_async_copy(frow2.at[h], acc.at[dst_v.at[0]],
                              ssems[h]).wait()

    def _start(b, k):
        pltpu.async_copy(ytab.at[gidx_v.at[b]], rows_i.at[k], sems[k])

    def _wait(b, k):
        pltpu.make_async_copy(ytab.at[gidx_v.at[b]], rows_i.at[k],
                              sems[k]).wait()

    for p in range(_NB_TILE // _NB_PHASE):
        row0 = wid * _NB_TILE + p * _NB_PHASE
        pltpu.sync_copy(gidx_hbm.at[pl.ds(row0, _NB_PHASE)], gidx_v)
        pltpu.sync_copy(dst_hbm.at[pl.ds(2 * row0, 2 * _NB_PHASE)], dst_v)

        for k in range(_DEPTH):
            _start(k, k)

        def _step(i, _):
            for k in range(_DEPTH):
                b = i * _DEPTH + k
                _wait(b, k)
                for h in range(2):
                    @pl.when(b > 0)
                    def _():
                        _wait_scatter(h)
                    _convert_half(k, h)
                    pltpu.async_copy(frow2.at[h], acc.at[dst_v.at[2 * b + h]],
                                     ssems[h], add=True)
                nxt = b + _DEPTH

                @pl.when(nxt < _NB_PHASE)
                def _():
                    _start(nxt, k)
            return 0
        jax.lax.fori_loop(0, _NB_PHASE // _DEPTH, _step, 0)
        for h in range(2):
            _wait_scatter(h)
    plsc.subcore_barrier()

    for cc, out_ref in ((0, out0), (1, out1)):
        @pl.when(c == cc)
        def _():
            @pl.when(s < _NS - 1)
            def _():
                r0 = pl.multiple_of(s * 624, 8)
                pltpu.sync_copy(acc.at[pl.ds(r0, 624)],
                                out_ref.at[pl.ds(r0, 624)])

            @pl.when(s == _NS - 1)
            def _():
                pltpu.sync_copy(acc.at[pl.ds(9360, 640)],
                                out_ref.at[pl.ds(9360, 640)])


def _sc_aggregate(ytab, gidx2d, dst2d):
    mesh = plsc.VectorSubcoreMesh(core_axis_name="c", subcore_axis_name="s")
    f = pl.kernel(
        _sc_body,
        out_type=[jax.ShapeDtypeStruct((_N, _D), jnp.float32),
                  jax.ShapeDtypeStruct((_N, _D), jnp.float32)],
        mesh=mesh,
        compiler_params=pltpu.CompilerParams(use_tc_tiling_on_sc=False,
                                             needs_layout_passes=False),
        scratch_types=[
            pltpu.VMEM_SHARED((_ACC_ROWS, _D), jnp.float32),
            pltpu.VMEM((_NB_PHASE, _B), jnp.int32),
            pltpu.VMEM((2 * _NB_PHASE, _HB), jnp.int32),
            pltpu.VMEM((_DEPTH, _B, _DW), jnp.int32),
            pltpu.VMEM((2, _HB, _D), jnp.float32),
        ] + [pltpu.SemaphoreType.DMA] * (_DEPTH + 2),
    )
    return f(ytab, gidx2d, dst2d)


@jax.jit
def kernel(x, edge_index, edge_type, W1, S1, b1, W2, S2, b2):
    src = edge_index[0].astype(jnp.int32)
    dst = edge_index[1].astype(jnp.int32)
    et = edge_type.astype(jnp.int32)

    pad = _E_PAD - _E
    src2d = jnp.concatenate(
        [src, jnp.zeros((pad,), jnp.int32)]).reshape(_E_PAD // _B, _B)
    et2d = jnp.concatenate(
        [et, jnp.zeros((pad,), jnp.int32)]).reshape(_E_PAD // _B, _B)
    dst2d = jnp.concatenate(
        [dst, jnp.full((pad,), _N, jnp.int32)]).reshape(_E_PAD // _HB, _HB)
    gidx2d = _make_gidx(src2d, et2d)

    b1r = b1.reshape(1, _D)
    b2r = b2.reshape(1, _D)

    self1 = _matmul_self(x, S1)
    ytab1 = _matmul_msgs(x, W1[:, :, jnp.array(_PERM)])
    agg0, agg1 = _sc_aggregate(ytab1, gidx2d, dst2d)
    h = _combine(self1, agg0, agg1, x, b1r, residual=True)

    self2 = _matmul_self(h, S2)
    ytab2 = _matmul_msgs(h, W2[:, :, jnp.array(_PERM)])
    agg0b, agg1b = _sc_aggregate(ytab2, gidx2d, dst2d)
    out = _combine(self2, agg0b, agg1b, h, b2r, residual=False)
    return out
